# baseline (speedup 1.0000x reference)
import functools

import jax
import jax.numpy as jnp
import numpy as np
from jax import lax
from jax.experimental import pallas as pl
from jax.experimental.pallas import tpu as pltpu
from jax.experimental.pallas import tpu_sc as plsc

_CHUNK = 128


def _tc_project(x, bases_perm, coeff, W_self, bias2d):
    N, D = x.shape
    R, B = coeff.shape
    NB = 400
    assert N % NB == 0

    def body(coeff_ref, x_ref, bases_ref, wself_ref, bias_ref, p_ref, s_ref):
        xb = x_ref[...]
        projs = [jnp.dot(xb, bases_ref[b], preferred_element_type=jnp.float32)
                 for b in range(B)]
        for r in range(R):
            acc = projs[0] * coeff_ref[r, 0]
            for b in range(1, B):
                acc = acc + projs[b] * coeff_ref[r, b]
            p_ref[:, r * D:(r + 1) * D] = acc.astype(jnp.bfloat16)
        s_ref[...] = (jnp.dot(xb, wself_ref[...],
                              preferred_element_type=jnp.float32)
                      + bias_ref[...])

    return pl.pallas_call(
        body,
        grid=(N // NB,),
        in_specs=[
            pl.BlockSpec(memory_space=pltpu.SMEM),
            pl.BlockSpec((NB, D), lambda i: (i, 0)),
            pl.BlockSpec((B, D, D), lambda i: (0, 0, 0)),
            pl.BlockSpec((D, D), lambda i: (0, 0)),
            pl.BlockSpec((1, D), lambda i: (0, 0)),
        ],
        out_specs=[
            pl.BlockSpec((NB, R * D), lambda i: (i, 0)),
            pl.BlockSpec((NB, D), lambda i: (i, 0)),
        ],
        out_shape=[
            jax.ShapeDtypeStruct((N, R * D), jnp.bfloat16),
            jax.ShapeDtypeStruct((N, D), jnp.float32),
        ],
    )(coeff, x, bases_perm, W_self, bias2d)


def _sc_edge_aggregate(p_words, meta3, R, NH, CH):
    info = plsc.get_sparse_core_info()
    NC, NS = info.num_cores, info.num_subcores
    DW = p_words.shape[1]
    D = DW * 2
    rows_per_tile = NH // NS
    n_drain = rows_per_tile // _CHUNK
    mesh = plsc.VectorSubcoreMesh(core_axis_name="c", subcore_axis_name="s")

    MDEPTH = 8

    @functools.partial(
        pl.kernel,
        out_type=jax.ShapeDtypeStruct((NC * NH, D), jnp.float32),
        mesh=mesh,
        scratch_types=[
            [pltpu.VMEM((3, _CHUNK), jnp.int32)] * MDEPTH,
            [pltpu.VMEM((_CHUNK,), jnp.int32)] * 2,
            [pltpu.VMEM((_CHUNK, DW), jnp.int32)] * 2,
            pltpu.VMEM((_CHUNK, D), jnp.float32),
            pltpu.VMEM_SHARED((NH, D), jnp.float32),
            [pltpu.SemaphoreType.DMA] * MDEPTH,
            [pltpu.SemaphoreType.DMA] * 2,
            pltpu.SemaphoreType.DMA,
        ],
        compiler_params=pltpu.CompilerParams(use_tc_tiling_on_sc=False),
    )
    def k(p_hbm, meta_hbm, out_hbm,
          metab, gidxb, rowsI, rowsF, h_sh, msem, gsem, ssem):
        cid = lax.axis_index("c")
        sid = lax.axis_index("s")
        wid = sid * NC + cid
        base_row = wid * CH

        def mfire(j, s):
            pltpu.async_copy(meta_hbm.at[base_row + j], metab[s], msem[s])

        def mwait(s):
            pltpu.make_async_copy(meta_hbm.at[0], metab[s], msem[s]).wait()

        def gidxc(g, s):
            for i in range(_CHUNK // 16):
                sl = pl.ds(i * 16, 16)
                gidxb[g][sl] = metab[s][0, sl] * R + metab[s][1, sl]

        def gfire(g):
            pltpu.async_copy(p_hbm.at[gidxb[g]], rowsI[g], gsem[g])

        def gwait(g):
            pltpu.make_async_copy(p_hbm.at[pl.ds(0, _CHUNK)], rowsI[g],
                                  gsem[g]).wait()

        def convert(g):
            himask = jnp.full((16,), -65536, jnp.int32)
            sh16 = jnp.full((16,), 16, jnp.int32)

            def crow(r, _):
                for w in range(DW // 16):
                    v = rowsI[g][r, pl.ds(w * 16, 16)]
                    lo = lax.bitcast_convert_type(
                        jnp.left_shift(v, sh16), jnp.float32)
                    hi = lax.bitcast_convert_type(
                        jnp.bitwise_and(v, himask), jnp.float32)
                    rowsF[r, pl.ds(32 * w, 16)] = lo
                    rowsF[r, pl.ds(32 * w + 16, 16)] = hi
                return 0
            lax.fori_loop(0, _CHUNK, crow, 0)

        def sfire(s):
            pltpu.async_copy(rowsF, h_sh.at[metab[s].at[2]], ssem, add=True)

        def swait():
            pltpu.make_async_copy(out_hbm.at[pl.ds(0, _CHUNK)], rowsF,
                                  ssem).wait()

        zero16 = jnp.zeros((16,), jnp.float32)
        nlane = D // 16

        def zrow(i, _):
            rowsF[i // nlane, pl.ds((i % nlane) * 16, 16)] = zero16
            return 0
        lax.fori_loop(0, _CHUNK * nlane, zrow, 0)

        stripe = sid * rows_per_tile

        def zcopy(t, _):
            pltpu.sync_copy(rowsF, h_sh.at[pl.ds(stripe + t * _CHUNK,
                                                 _CHUNK)])
            return 0
        lax.fori_loop(0, n_drain, zcopy, 0)
        plsc.subcore_barrier()

        T = CH // MDEPTH
        assert CH % MDEPTH == 0 and T >= 2

        def chunk_step(c, q, first_octet, last_octet):
            g, m = q % 2, q
            gn, mn = (q + 1) % 2, (q + 1) % MDEPTH
            mp = (q - 1) % MDEPTH
            has_next = not (last_octet and q == MDEPTH - 1)
            if has_next:
                mwait(mn)
                gidxc(gn, mn)
                gfire(gn)
            gwait(g)
            if not (first_octet and q == 0):
                swait()
                if (not last_octet) or q == 0:
                    mfire(c - 1 + MDEPTH, mp)
            convert(g)
            sfire(m)

        for s in range(MDEPTH):
            mfire(s, s)
        mwait(0)
        gidxc(0, 0)
        gfire(0)

        for q in range(MDEPTH):
            chunk_step(q, q, True, T == 1)
        if T > 2:
            def octet(u, _):
                for q in range(MDEPTH):
                    chunk_step(u * MDEPTH + q, q, False, False)
                return 0
            lax.fori_loop(1, T - 1, octet, 0)
        if T > 1:
            for q in range(MDEPTH):
                chunk_step((T - 1) * MDEPTH + q, q, False, True)
        swait()
        plsc.subcore_barrier()

        def dcopy(t, _):
            base = stripe + t * _CHUNK
            pltpu.sync_copy(h_sh.at[pl.ds(base, _CHUNK)], rowsF)
            pltpu.sync_copy(rowsF, out_hbm.at[pl.ds(cid * NH + base,
                                                    _CHUNK)])
            return 0
        lax.fori_loop(0, n_drain, dcopy, 0)

    return k(p_words, meta3)


def _tc_combine(a, b, s):
    N, D = a.shape
    NB = 400

    def body(a_ref, b_ref, s_ref, o_ref):
        o_ref[...] = a_ref[...] + b_ref[...] + s_ref[...]

    spec = pl.BlockSpec((NB, D), lambda i: (i, 0))
    return pl.pallas_call(
        body,
        grid=(N // NB,),
        in_specs=[spec, spec, spec],
        out_specs=spec,
        out_shape=jax.ShapeDtypeStruct((N, D), jnp.float32),
    )(a, b, s)


def kernel(x, edge_index, etypes, bases, coeff, W_self, bias):
    N, D = x.shape
    E = edge_index.shape[1]
    R, B = coeff.shape

    info = plsc.get_sparse_core_info()
    NC, NS = info.num_cores, info.num_subcores
    NW = NC * NS

    NH = ((N + 1 + NS * _CHUNK - 1) // (NS * _CHUNK)) * (NS * _CHUNK)
    CH = (E + NW * _CHUNK - 1) // (NW * _CHUNK)
    CH = max(((CH + 7) // 8) * 8, 16)
    E_pad = NW * CH * _CHUNK

    sigma = np.empty((D,), np.int32)
    for g0 in range(D // 32):
        for t in range(16):
            sigma[32 * g0 + 2 * t] = 32 * g0 + t
            sigma[32 * g0 + 2 * t + 1] = 32 * g0 + 16 + t
    bases_perm = bases[:, :, sigma]

    p16, self_term = _tc_project(x, bases_perm, coeff, W_self,
                                 bias.reshape(1, D))
    p_words = jax.lax.bitcast_convert_type(
        p16.reshape(N * R, D // 2, 2), jnp.int32)

    src = edge_index[0]
    dst = edge_index[1]
    pad = E_pad - E
    src2 = jnp.concatenate([src, jnp.zeros((pad,), jnp.int32)])
    et2 = jnp.concatenate([etypes, jnp.zeros((pad,), jnp.int32)])
    dst2 = jnp.concatenate([dst, jnp.full((pad,), N, jnp.int32)])
    meta3 = jnp.stack([src2.reshape(-1, _CHUNK), et2.reshape(-1, _CHUNK),
                       dst2.reshape(-1, _CHUNK)], axis=1)

    partial = _sc_edge_aggregate(p_words, meta3, R, NH, CH)
    partial = partial.reshape(NC, NH, D)

    return _tc_combine(partial[0, :N], partial[1, :N], self_term)


# speedup vs baseline: 12.9787x; 12.9787x over previous
import functools

import jax
import jax.numpy as jnp
from jax import lax
from jax.experimental import pallas as pl
from jax.experimental.pallas import tpu as pltpu
from jax.experimental.pallas import tpu_sc as plsc

_CHUNK = 128
_MDEPTH = 8
_FRAC0 = 0.40


def _tc_project(x, bases, coeff, W_self, bias2d):
    N, D = x.shape
    R, B = coeff.shape
    NB = 400
    assert N % NB == 0

    def body(coeff_ref, x_ref, bases_ref, wself_ref, bias_ref, p_ref, s_ref):
        xb = x_ref[...]
        projs = [jnp.dot(xb, bases_ref[b], preferred_element_type=jnp.float32)
                 for b in range(B)]
        for r in range(R):
            acc = projs[0] * coeff_ref[r, 0]
            for b in range(1, B):
                acc = acc + projs[b] * coeff_ref[r, b]
            p_ref[:, r * D:(r + 1) * D] = acc
        s_ref[...] = (jnp.dot(xb, wself_ref[...],
                              preferred_element_type=jnp.float32)
                      + bias_ref[...])

    return pl.pallas_call(
        body,
        grid=(N // NB,),
        in_specs=[
            pl.BlockSpec(memory_space=pltpu.SMEM),
            pl.BlockSpec((NB, D), lambda i: (i, 0)),
            pl.BlockSpec((B, D, D), lambda i: (0, 0, 0)),
            pl.BlockSpec((D, D), lambda i: (0, 0)),
            pl.BlockSpec((1, D), lambda i: (0, 0)),
        ],
        out_specs=[
            pl.BlockSpec((NB, R * D), lambda i: (i, 0)),
            pl.BlockSpec((NB, D), lambda i: (i, 0)),
        ],
        out_shape=[
            jax.ShapeDtypeStruct((N, R * D), jnp.float32),
            jax.ShapeDtypeStruct((N, D), jnp.float32),
        ],
    )(coeff, x, bases, W_self, bias2d)


def _sc_edge_aggregate(p_flat, meta3, R, NH, CH, CH0, CH1):
    info = plsc.get_sparse_core_info()
    NC, NS = info.num_cores, info.num_subcores
    D = p_flat.shape[1]
    rows_per_tile = NH // NS
    n_drain = rows_per_tile // _CHUNK
    mesh = plsc.VectorSubcoreMesh(core_axis_name="c", subcore_axis_name="s")

    @functools.partial(
        pl.kernel,
        out_type=jax.ShapeDtypeStruct((NC * NH, D), jnp.float32),
        mesh=mesh,
        scratch_types=[
            [pltpu.VMEM((3, _CHUNK), jnp.int32)] * _MDEPTH,
            [pltpu.VMEM((_CHUNK,), jnp.int32)] * 2,
            [pltpu.VMEM((_CHUNK, D), jnp.float32)] * 2,
            pltpu.VMEM_SHARED((NH, D), jnp.float32),
            [pltpu.SemaphoreType.DMA] * _MDEPTH,
            [pltpu.SemaphoreType.DMA] * 2,
            [pltpu.SemaphoreType.DMA] * 2,
        ],
    )
    def k(p_hbm, meta_hbm, out_hbm,
          metab, gidxb, rows, h_sh, msem, gsem, ssem):
        cid = lax.axis_index("c")
        sid = lax.axis_index("s")
        wid = sid * NC + cid
        base_row = wid * CH
        climit = jnp.where(cid == 0, CH0, CH1)

        def active(c):
            return c < climit

        def mfire(j, s):
            pltpu.async_copy(meta_hbm.at[base_row + j], metab[s], msem[s])

        def mwait(s):
            pltpu.make_async_copy(meta_hbm.at[0], metab[s], msem[s]).wait()

        def gidxc(g, s):
            for i in range(_CHUNK // 16):
                sl = pl.ds(i * 16, 16)
                gidxb[g][sl] = metab[s][0, sl] * R + metab[s][1, sl]

        def gfire(g):
            pltpu.async_copy(p_hbm.at[gidxb[g]], rows[g], gsem[g])

        def gwait(g):
            pltpu.make_async_copy(p_hbm.at[pl.ds(0, _CHUNK)], rows[g],
                                  gsem[g]).wait()

        def sfire(g, s):
            pltpu.async_copy(rows[g], h_sh.at[metab[s].at[2]], ssem[g],
                             add=True)

        def swait(g):
            pltpu.make_async_copy(p_hbm.at[pl.ds(0, _CHUNK)], rows[g],
                                  ssem[g]).wait()

        zero16 = jnp.zeros((16,), jnp.float32)
        nlane = D // 16

        def zrow(i, _):
            rows[0][i // nlane, pl.ds((i % nlane) * 16, 16)] = zero16
            return 0
        lax.fori_loop(0, _CHUNK * nlane, zrow, 0)

        stripe = sid * rows_per_tile

        def zcopy(t, _):
            pltpu.sync_copy(rows[0], h_sh.at[pl.ds(stripe + t * _CHUNK,
                                                   _CHUNK)])
            return 0
        lax.fori_loop(0, n_drain, zcopy, 0)
        plsc.subcore_barrier()

        T = CH // _MDEPTH
        assert CH % _MDEPTH == 0 and T >= 2

        def chunk_step(c, q, first_octet, last_octet):
            g, m = q % 2, q
            gn, mn = (q + 1) % 2, (q + 1) % _MDEPTH
            mp = (q - 1) % _MDEPTH
            has_next = not (last_octet and q == _MDEPTH - 1)
            if has_next:
                mwait(mn)
                gidxc(gn, mn)
            if not (first_octet and q == 0):
                @pl.when(active(c - 1))
                def _():
                    swait(gn)
                if (not last_octet) or q == 0:
                    mfire(c - 1 + _MDEPTH, mp)
            if has_next:
                @pl.when(active(c + 1))
                def _():
                    gfire(gn)

            @pl.when(active(c))
            def _():
                gwait(g)
                sfire(g, m)

        for s in range(_MDEPTH):
            mfire(s, s)
        mwait(0)
        gidxc(0, 0)

        @pl.when(active(0))
        def _():
            gfire(0)

        for q in range(_MDEPTH):
            chunk_step(q, q, True, T == 1)
        if T > 2:
            def octet(u, _):
                for q in range(_MDEPTH):
                    chunk_step(u * _MDEPTH + q, q, False, False)
                return 0
            lax.fori_loop(1, T - 1, octet, 0)
        if T > 1:
            for q in range(_MDEPTH):
                chunk_step((T - 1) * _MDEPTH + q, q, False, True)

        @pl.when(active(CH - 1))
        def _():
            swait((CH - 1) % 2)
        plsc.subcore_barrier()

        def dcopy(t, _):
            base = stripe + t * _CHUNK
            pltpu.sync_copy(h_sh.at[pl.ds(base, _CHUNK)], rows[0])
            pltpu.sync_copy(rows[0], out_hbm.at[pl.ds(cid * NH + base,
                                                      _CHUNK)])
            return 0
        lax.fori_loop(0, n_drain, dcopy, 0)

    return k(p_flat, meta3)


def _tc_combine(a, b, s):
    N, D = a.shape
    NB = 400

    def body(a_ref, b_ref, s_ref, o_ref):
        o_ref[...] = a_ref[...] + b_ref[...] + s_ref[...]

    spec = pl.BlockSpec((NB, D), lambda i: (i, 0))
    return pl.pallas_call(
        body,
        grid=(N // NB,),
        in_specs=[spec, spec, spec],
        out_specs=spec,
        out_shape=jax.ShapeDtypeStruct((N, D), jnp.float32),
    )(a, b, s)


def kernel(x, edge_index, etypes, bases, coeff, W_self, bias):
    N, D = x.shape
    E = edge_index.shape[1]
    R, B = coeff.shape

    info = plsc.get_sparse_core_info()
    NC, NS = info.num_cores, info.num_subcores
    NW = NC * NS

    NH = ((N + 1 + NS * _CHUNK - 1) // (NS * _CHUNK)) * (NS * _CHUNK)

    CT = (E + _CHUNK - 1) // _CHUNK
    CH0 = max(1, int(round(CT * _FRAC0 / NS)))
    CH1 = (CT - NS * CH0 + NS - 1) // NS
    assert CH1 >= 1
    CH = max(((max(CH0, CH1) + _MDEPTH - 1) // _MDEPTH) * _MDEPTH,
             2 * _MDEPTH)

    src = edge_index[0]
    dst = edge_index[1]

    cap = [(CH0 if w % NC == 0 else CH1) * _CHUNK for w in range(NW)]
    total_cap = sum(cap)
    srcp = jnp.concatenate([src, jnp.zeros((total_cap - E,), jnp.int32)])
    etp = jnp.concatenate([etypes, jnp.zeros((total_cap - E,), jnp.int32)])
    dstp = jnp.concatenate([dst, jnp.full((total_cap - E,), N, jnp.int32)])
    segs = []
    off = 0
    pad_c = CH * _CHUNK
    for w in range(NW):
        seg = jnp.stack([srcp[off:off + cap[w]], etp[off:off + cap[w]],
                         dstp[off:off + cap[w]]])
        segs.append(jnp.pad(seg, ((0, 0), (0, pad_c - cap[w]))))
        off += cap[w]
    meta = jnp.stack(segs)
    meta = meta.reshape(NW, 3, CH, _CHUNK).transpose(0, 2, 1, 3)
    meta3 = meta.reshape(NW * CH, 3, _CHUNK)

    p, self_term = _tc_project(x, bases, coeff, W_self, bias.reshape(1, D))
    p_flat = p.reshape(N * R, D)

    partial = _sc_edge_aggregate(p_flat, meta3, R, NH, CH, CH0, CH1)
    partial = partial.reshape(NC, NH, D)

    return _tc_combine(partial[0, :N], partial[1, :N], self_term)
